# Initial kernel scaffold; baseline (speedup 1.0000x reference)
import jax
import jax.numpy as jnp
from jax.experimental import pallas as pl


def kernel(pred, target, smoothing):
    raise NotImplementedError("write your pallas kernel here")


# speedup vs baseline: 1.8980x; 1.8980x over previous
import functools

import jax
import jax.numpy as jnp
from jax import lax
from jax.experimental import pallas as pl
from jax.experimental.pallas import tpu as pltpu
from jax.experimental.pallas import tpu_sc as plsc

BATCH = 16384
CLASSES = 1000
LANES = 16
NW = 32
BPW = BATCH // NW
GCHUNK = 128
NCHUNK = BPW // GCHUNK
ROW_BLOCK = 1024


def _sc_sparse_part(pred16, target, smooth):
    mesh = plsc.VectorSubcoreMesh(core_axis_name="c", subcore_axis_name="s")

    @functools.partial(
        pl.kernel,
        mesh=mesh,
        compiler_params=pltpu.CompilerParams(
            use_tc_tiling_on_sc=False, needs_layout_passes=False),
        out_type=jax.ShapeDtypeStruct((NW, LANES), jnp.float32),
        scratch_types=(
            [pltpu.VMEM((BPW,), jnp.int32),
             pltpu.VMEM((BPW,), jnp.float32),
             pltpu.VMEM((BPW,), jnp.int32)]
            + [pltpu.VMEM((GCHUNK,), jnp.int32) for _ in range(NCHUNK)]
            + [pltpu.VMEM((GCHUNK, LANES), jnp.float32) for _ in range(NCHUNK)]
            + [pltpu.VMEM((LANES,), jnp.float32),
               pltpu.SemaphoreType.DMA]
        ),
    )
    def body(pred_hbm, tgt_hbm, sm_hbm, out_hbm,
             tgt_v, sm_v, rem_v, *rest):
        gidx = rest[:NCHUNK]
        rows = rest[NCHUNK:2 * NCHUNK]
        acc_v, sem = rest[2 * NCHUNK], rest[2 * NCHUNK + 1]
        wid = lax.axis_index("s") * 2 + lax.axis_index("c")
        base = wid * BPW
        pltpu.sync_copy(tgt_hbm.at[pl.ds(base, BPW)], tgt_v)
        pltpu.sync_copy(sm_hbm.at[pl.ds(base, BPW)], sm_v)
        lane = lax.iota(jnp.int32, LANES)
        per_chunk = GCHUNK // LANES
        for c in range(BPW // LANES):
            t = tgt_v[pl.ds(c * LANES, LANES)]
            flat = (base + c * LANES + lane) * CLASSES + t
            gidx[c // per_chunk][pl.ds((c % per_chunk) * LANES, LANES)] = (
                jnp.right_shift(flat, 4))
            rem_v[pl.ds(c * LANES, LANES)] = jnp.bitwise_and(flat, LANES - 1)
        copies = [
            pltpu.async_copy(pred_hbm.at[gidx[k]], rows[k], sem)
            for k in range(NCHUNK)
        ]
        for cp in copies:
            cp.wait()
        acc = jnp.zeros((LANES,), jnp.float32)
        for c in range(BPW // LANES):
            p = plsc.load_gather(
                rows[c // per_chunk],
                [(c % per_chunk) * LANES + lane,
                 rem_v[pl.ds(c * LANES, LANES)]])
            s = jnp.maximum(sm_v[pl.ds(c * LANES, LANES)], 0.1)
            acc = acc + (s * (1.0 / (CLASSES - 1)) + s - 1.0) * p
        acc_v[...] = acc
        pltpu.sync_copy(acc_v, out_hbm.at[wid])

    return body(pred16, target, smooth)


def _tc_dense_part(pred, smooth, sc_part):
    def body(pred_ref, sm_ref, scp_ref, out_ref):
        i = pl.program_id(0)
        x = pred_ref[...]
        s = jnp.maximum(sm_ref[...], 0.1)
        fill = s * (1.0 / (CLASSES - 1))
        partial = -jnp.sum(fill * jnp.sum(x, axis=1, keepdims=True))

        @pl.when(i == 0)
        def _():
            out_ref[0, 0] = jnp.sum(scp_ref[...])

        out_ref[0, 0] += partial

    return pl.pallas_call(
        body,
        grid=(BATCH // ROW_BLOCK,),
        in_specs=[
            pl.BlockSpec((ROW_BLOCK, CLASSES), lambda i: (i, 0)),
            pl.BlockSpec((ROW_BLOCK, 1), lambda i: (i, 0)),
            pl.BlockSpec((NW, LANES), lambda i: (0, 0)),
        ],
        out_specs=pl.BlockSpec(memory_space=pltpu.SMEM),
        out_shape=jax.ShapeDtypeStruct((1, 1), jnp.float32),
    )(pred, smooth, sc_part)


def kernel(pred, target, smoothing):
    pred16 = pred.reshape(BATCH * CLASSES // LANES, LANES)
    sm = smoothing.reshape(BATCH)
    scp = _sc_sparse_part(pred16, target, sm)
    out = _tc_dense_part(pred, smoothing, scp)
    return out[0, 0]
